# baseline (speedup 1.0000x reference)
import dataclasses
import functools

import jax
import jax.numpy as jnp
from jax import lax
from jax.experimental import pallas as pl
from jax.experimental.pallas import tpu as pltpu
from jax.experimental.pallas import tpu_sc as plsc

N = 10000
E = 320000
H = 128
L = 3
G = 128

NC = 2
NS = 16
LANES = 16
NW = NC * NS
EPW = E // NW
CHUNK = 80
NCHUNK = EPW // CHUNK
NPAD = 10112
RPT = NPAD // NS

_f32 = jnp.float32


def _edge_stage_body(h_hbm, ep_hbm, src_hbm, dst_hbm, zeros_hbm, out_hbm,
                     sidx, didx, rows, ebuf, sbuf, aggr_sh, *sems):
    cid = lax.axis_index("c")
    sid = lax.axis_index("s")
    wid = cid * NS + sid
    gsem = sems[0:2]
    esem = sems[2]
    ssem = sems[3]
    isem_s = sems[4:6]
    isem_d = sems[6:8]

    pltpu.sync_copy(zeros_hbm.at[pl.ds(sid * RPT, RPT)],
                    aggr_sh.at[pl.ds(sid * RPT, RPT)])

    base0 = wid * EPW

    def issue_e(ci):
        pltpu.async_copy(ep_hbm.at[pl.ds(base0 + ci * CHUNK, CHUNK)],
                         ebuf, esem)

    def issue_gather(ci, b):
        pltpu.async_copy(h_hbm.at[sidx.at[b]], rows.at[b], gsem[b])

    pltpu.sync_copy(src_hbm.at[pl.ds(base0, CHUNK)], sidx.at[0])
    pltpu.sync_copy(src_hbm.at[pl.ds(base0 + CHUNK, CHUNK)], sidx.at[1])
    pltpu.async_copy(dst_hbm.at[pl.ds(base0, CHUNK)], didx.at[0], isem_d[0])
    plsc.subcore_barrier()
    issue_gather(0, 0)
    issue_gather(1, 1)
    issue_e(0)

    def step(ci, b, first, issue_d, issue_next):
        pltpu.make_async_copy(h_hbm.at[sidx.at[b]], rows.at[b],
                              gsem[b]).wait()
        pltpu.make_async_copy(ep_hbm.at[pl.ds(base0 + ci * CHUNK, CHUNK)],
                              ebuf, esem).wait()
        if issue_next:
            pltpu.async_copy(src_hbm.at[pl.ds(base0 + (ci + 2) * CHUNK, CHUNK)],
                             sidx.at[b], isem_s[b])
        if not first:
            pltpu.make_async_copy(sbuf, aggr_sh.at[didx.at[1 - b]],
                                  ssem).wait()
        if issue_d:
            pltpu.async_copy(dst_hbm.at[pl.ds(base0 + (ci + 1) * CHUNK, CHUNK)],
                             didx.at[1 - b], isem_d[1 - b])

        rows_b = rows.at[b]

        @pl.loop(0, CHUNK)
        def _(r):
            for g in range(0, H, LANES):
                sbuf[r, pl.ds(g, LANES)] = jnp.maximum(
                    rows_b[r, pl.ds(g, LANES)] + ebuf[r, pl.ds(g, LANES)], 0.0)

        if issue_d:
            issue_e(ci + 1)
        pltpu.make_async_copy(dst_hbm.at[pl.ds(base0, CHUNK)], didx.at[b],
                              isem_d[b]).wait()
        pltpu.async_copy(sbuf, aggr_sh.at[didx.at[b]], ssem, add=True)
        if issue_next:
            pltpu.make_async_copy(src_hbm.at[pl.ds(base0, CHUNK)], sidx.at[b],
                                  isem_s[b]).wait()
            issue_gather(ci + 2, b)

    step(0, 0, True, True, True)
    step(1, 1, False, True, True)

    @pl.loop(2, NCHUNK - 3, step=2)
    def _(k):
        step(k, 0, False, True, True)
        step(k + 1, 1, False, True, True)

    step(NCHUNK - 3, 0, False, True, True)
    step(NCHUNK - 2, 1, False, True, False)
    step(NCHUNK - 1, 0, False, False, False)

    pltpu.make_async_copy(sbuf, aggr_sh.at[didx.at[0]], ssem).wait()

    plsc.subcore_barrier()
    pltpu.sync_copy(aggr_sh.at[pl.ds(sid * RPT, RPT)],
                    out_hbm.at[cid, pl.ds(sid * RPT, RPT)])


def _edge_stage(h, ep, src, dst, zeros):
    mesh = plsc.VectorSubcoreMesh(core_axis_name="c", subcore_axis_name="s",
                                  num_cores=NC, num_subcores=NS)
    k = pl.kernel(
        _edge_stage_body,
        out_type=jax.ShapeDtypeStruct((NC, NPAD, H), _f32),
        mesh=mesh,
        scratch_types=[
            pltpu.VMEM((2, CHUNK), jnp.int32),
            pltpu.VMEM((2, CHUNK), jnp.int32),
            pltpu.VMEM((2, CHUNK, H), _f32),
            pltpu.VMEM((CHUNK, H), _f32),
            pltpu.VMEM((CHUNK, H), _f32),
            pltpu.VMEM_SHARED((NPAD, H), _f32),
        ] + [pltpu.SemaphoreType.DMA] * 8,
    )
    return k(h, ep, src, dst, zeros)


def _pack_bf16_pairs(x):
    lo = lax.bitcast_convert_type(
        x[:, :H // 2].astype(jnp.bfloat16), jnp.uint16).astype(jnp.uint32)
    hi = lax.bitcast_convert_type(
        x[:, H // 2:].astype(jnp.bfloat16), jnp.uint16).astype(jnp.uint32)
    return lax.bitcast_convert_type(lo | (hi << 16), jnp.int32)


def _node_encode_body(x_ref, w_ref, b_ref, o_ref):
    o_ref[...] = (jnp.dot(x_ref[...], w_ref[...],
                          preferred_element_type=_f32) + b_ref[...])


def _edge_encode_body(ea_ref, w1_ref, b1_ref, w2_ref, b2_ref, o_ref):
    dims = (((0,), (0,)), ((), ()))
    t = lax.dot_general(ea_ref[...], w1_ref[...], dims,
                        preferred_element_type=_f32)
    t = jnp.maximum(t + b1_ref[...], 0.0)
    o_ref[...] = (jnp.dot(t, w2_ref[...], preferred_element_type=_f32)
                  + b2_ref[...])


def _node_mlp_body(h_ref, a0_ref, a1_ref, w1_ref, b1_ref, w2_ref, b2_ref,
                   o_ref):
    z = h_ref[...] + a0_ref[...] + a1_ref[...]
    t = jnp.dot(z, w1_ref[...], preferred_element_type=_f32)
    t = jnp.maximum(t + b1_ref[...], 0.0)
    t = jnp.dot(t, w2_ref[...], preferred_element_type=_f32) + b2_ref[...]
    o_ref[...] = jnp.maximum(t, 0.0)


def _pool_heads_body(h_ref, batch_ref, cw1_ref, cb1_ref, cw2_ref, cb2_ref,
                     rw1_ref, rb1_ref, rw2_ref, rb2_ref, s_ref, r_ref):
    b2d = batch_ref[...]
    gids = lax.broadcasted_iota(jnp.int32, (N, G), 1)
    p = (b2d == gids).astype(_f32)
    dims = (((0,), (0,)), ((), ()))
    sums = lax.dot_general(p, h_ref[...], dims,
                           preferred_element_type=_f32)
    counts = lax.dot_general(p, jnp.ones((N, 1), _f32), dims,
                             preferred_element_type=_f32)
    g = sums / jnp.maximum(counts, 1.0)
    cs = jnp.maximum(jnp.dot(g, cw1_ref[...], preferred_element_type=_f32)
                     + cb1_ref[...], 0.0)
    s_ref[...] = (jnp.dot(cs, cw2_ref[...], preferred_element_type=_f32)
                  + cb2_ref[...])
    rs = jnp.maximum(jnp.dot(g, rw1_ref[...], preferred_element_type=_f32)
                     + rb1_ref[...], 0.0)
    r_ref[...] = (jnp.dot(rs, rw2_ref[...], preferred_element_type=_f32)
                  + rb2_ref[...])


def _node_encode(x, w, b):
    return pl.pallas_call(
        _node_encode_body,
        out_shape=jax.ShapeDtypeStruct((N, H), _f32),
    )(x, w, b)


_BE = 2560


def _edge_encode(ea, w1, b1, w2, b2):
    nblk = E // _BE
    return pl.pallas_call(
        _edge_encode_body,
        grid=(nblk,),
        in_specs=[
            pl.BlockSpec((3, _BE), lambda i: (0, i)),
            pl.BlockSpec((3, H), lambda i: (0, 0)),
            pl.BlockSpec((1, H), lambda i: (0, 0)),
            pl.BlockSpec((H, H), lambda i: (0, 0)),
            pl.BlockSpec((1, H), lambda i: (0, 0)),
        ],
        out_specs=pl.BlockSpec((_BE, H), lambda i: (i, 0)),
        out_shape=jax.ShapeDtypeStruct((E, H), _f32),
    )(ea, w1, b1, w2, b2)


_BN = 2000


def _node_mlp(h, a0, a1, w1, b1, w2, b2):
    nblk = N // _BN
    return pl.pallas_call(
        _node_mlp_body,
        grid=(nblk,),
        in_specs=[
            pl.BlockSpec((_BN, H), lambda i: (i, 0)),
            pl.BlockSpec((_BN, H), lambda i: (i, 0)),
            pl.BlockSpec((_BN, H), lambda i: (i, 0)),
            pl.BlockSpec((H, H), lambda i: (0, 0)),
            pl.BlockSpec((1, H), lambda i: (0, 0)),
            pl.BlockSpec((H, H), lambda i: (0, 0)),
            pl.BlockSpec((1, H), lambda i: (0, 0)),
        ],
        out_specs=pl.BlockSpec((_BN, H), lambda i: (i, 0)),
        out_shape=jax.ShapeDtypeStruct((N, H), _f32),
    )(h, a0, a1, w1, b1, w2, b2)


def _pool_heads(h, batch2d, cw1, cb1, cw2, cb2, rw1, rb1, rw2, rb2):
    return pl.pallas_call(
        _pool_heads_body,
        out_shape=(jax.ShapeDtypeStruct((G, 1), _f32),
                   jax.ShapeDtypeStruct((G, 1), _f32)),
    )(h, batch2d, cw1, cb1, cw2, cb2, rw1, rb1, rw2, rb2)


def kernel(x, edge_index, edge_attr, batch,
           ne_w, ne_b, ee_w1, ee_b1, ee_w2, ee_b2,
           conv_w1, conv_b1, conv_w2, conv_b2,
           ch_w1, ch_b1, ch_w2, ch_b2,
           rh_w1, rh_b1, rh_w2, rh_b2):
    src = edge_index[0]
    dst = edge_index[1]
    zeros = jnp.zeros((NPAD, H), _f32)

    h = _node_encode(x, ne_w, ne_b.reshape(1, H))
    ep = _edge_encode(edge_attr.T, ee_w1, ee_b1.reshape(1, H),
                      ee_w2, ee_b2.reshape(1, H))

    for i in range(L):
        parts = _edge_stage(h, ep, src, dst, zeros)
        h = _node_mlp(h, parts[0, :N], parts[1, :N],
                      conv_w1[i], conv_b1[i].reshape(1, H),
                      conv_w2[i], conv_b2[i].reshape(1, H))

    s, r = _pool_heads(h, batch.reshape(N, 1),
                       ch_w1, ch_b1.reshape(1, 64), ch_w2, ch_b2.reshape(1, 1),
                       rh_w1, rh_b1.reshape(1, 64), rh_w2, rh_b2.reshape(1, 1))
    return (s.reshape(G), r.reshape(G))


# speedup vs baseline: 1.1123x; 1.1123x over previous
import dataclasses
import functools

import jax
import jax.numpy as jnp
from jax import lax
from jax.experimental import pallas as pl
from jax.experimental.pallas import tpu as pltpu
from jax.experimental.pallas import tpu_sc as plsc

N = 10000
E = 320000
H = 128
L = 3
G = 128

NC = 2
NS = 16
LANES = 16
NW = NC * NS
EPW = E // NW
CHUNK = 80
NCHUNK = EPW // CHUNK
NPAD = 10112
RPT = NPAD // NS

_f32 = jnp.float32


def _edge_stage_body(h_hbm, ep_hbm, src_hbm, dst_hbm, zeros_hbm, out_hbm,
                     sidx, didx, rows, ebuf, sbuf, aggr_sh, *sems):
    cid = lax.axis_index("c")
    sid = lax.axis_index("s")
    wid = cid * NS + sid
    gsem = sems[0:2]
    esem = sems[2]
    ssem = sems[3]
    isem_s = sems[4:6]
    isem_d = sems[6:8]

    pltpu.sync_copy(zeros_hbm.at[pl.ds(sid * RPT, RPT)],
                    aggr_sh.at[pl.ds(sid * RPT, RPT)])

    base0 = wid * EPW

    def issue_e(ci):
        pltpu.async_copy(ep_hbm.at[pl.ds(base0 + ci * CHUNK, CHUNK)],
                         ebuf, esem)

    def issue_gather(ci, b):
        pltpu.async_copy(h_hbm.at[sidx.at[b]], rows.at[b], gsem[b])

    pltpu.sync_copy(src_hbm.at[pl.ds(base0, CHUNK)], sidx.at[0])
    pltpu.sync_copy(src_hbm.at[pl.ds(base0 + CHUNK, CHUNK)], sidx.at[1])
    pltpu.async_copy(dst_hbm.at[pl.ds(base0, CHUNK)], didx.at[0], isem_d[0])
    plsc.subcore_barrier()
    issue_gather(0, 0)
    issue_gather(1, 1)
    issue_e(0)

    def step(ci, b, first, issue_d, issue_next):
        pltpu.make_async_copy(h_hbm.at[sidx.at[b]], rows.at[b],
                              gsem[b]).wait()
        pltpu.make_async_copy(ep_hbm.at[pl.ds(base0 + ci * CHUNK, CHUNK)],
                              ebuf, esem).wait()
        if issue_next:
            pltpu.async_copy(src_hbm.at[pl.ds(base0 + (ci + 2) * CHUNK, CHUNK)],
                             sidx.at[b], isem_s[b])
        if issue_d:
            pltpu.async_copy(dst_hbm.at[pl.ds(base0 + (ci + 1) * CHUNK, CHUNK)],
                             didx.at[1 - b], isem_d[1 - b])

        rows_b = rows.at[b]

        @pl.loop(0, CHUNK)
        def _(r):
            for g in range(0, H, LANES):
                slc = (pl.ds(r, 1), pl.ds(g, LANES))
                v = rows_b.at[*slc][...] + ebuf.at[*slc][...]
                sbuf.at[*slc][...] = jnp.maximum(v, 0.0)

        if issue_d:
            issue_e(ci + 1)
        pltpu.make_async_copy(dst_hbm.at[pl.ds(base0, CHUNK)], didx.at[b],
                              isem_d[b]).wait()
        pltpu.async_copy(sbuf, aggr_sh.at[didx.at[b]], ssem, add=True)
        pltpu.make_async_copy(sbuf, aggr_sh.at[didx.at[b]], ssem).wait()
        if issue_next:
            pltpu.make_async_copy(src_hbm.at[pl.ds(base0, CHUNK)], sidx.at[b],
                                  isem_s[b]).wait()
            issue_gather(ci + 2, b)

    step(0, 0, True, True, True)
    step(1, 1, False, True, True)

    @pl.loop(2, NCHUNK - 3, step=2)
    def _(k):
        step(k, 0, False, True, True)
        step(k + 1, 1, False, True, True)

    step(NCHUNK - 3, 0, False, True, True)
    step(NCHUNK - 2, 1, False, True, False)
    step(NCHUNK - 1, 0, False, False, False)

    plsc.subcore_barrier()
    pltpu.sync_copy(aggr_sh.at[pl.ds(sid * RPT, RPT)],
                    out_hbm.at[cid, pl.ds(sid * RPT, RPT)])


def _edge_stage(h, ep, src, dst, zeros):
    mesh = plsc.VectorSubcoreMesh(core_axis_name="c", subcore_axis_name="s",
                                  num_cores=NC, num_subcores=NS)
    k = pl.kernel(
        _edge_stage_body,
        out_type=jax.ShapeDtypeStruct((NC, NPAD, H), _f32),
        mesh=mesh,
        scratch_types=[
            pltpu.VMEM((2, CHUNK), jnp.int32),
            pltpu.VMEM((2, CHUNK), jnp.int32),
            pltpu.VMEM((2, CHUNK, H), _f32),
            pltpu.VMEM((CHUNK, H), _f32),
            pltpu.VMEM((CHUNK, H), _f32),
            pltpu.VMEM_SHARED((NPAD, H), _f32),
        ] + [pltpu.SemaphoreType.DMA] * 8,
    )
    return k(h, ep, src, dst, zeros)


def _pack_bf16_pairs(x):
    lo = lax.bitcast_convert_type(
        x[:, :H // 2].astype(jnp.bfloat16), jnp.uint16).astype(jnp.uint32)
    hi = lax.bitcast_convert_type(
        x[:, H // 2:].astype(jnp.bfloat16), jnp.uint16).astype(jnp.uint32)
    return lax.bitcast_convert_type(lo | (hi << 16), jnp.int32)


def _node_encode_body(x_ref, w_ref, b_ref, o_ref):
    o_ref[...] = (jnp.dot(x_ref[...], w_ref[...],
                          preferred_element_type=_f32) + b_ref[...])


def _edge_encode_body(ea_ref, w1_ref, b1_ref, w2_ref, b2_ref, o_ref):
    dims = (((0,), (0,)), ((), ()))
    t = lax.dot_general(ea_ref[...], w1_ref[...], dims,
                        preferred_element_type=_f32)
    t = jnp.maximum(t + b1_ref[...], 0.0)
    o_ref[...] = (jnp.dot(t, w2_ref[...], preferred_element_type=_f32)
                  + b2_ref[...])


def _node_mlp_body(h_ref, a0_ref, a1_ref, w1_ref, b1_ref, w2_ref, b2_ref,
                   o_ref):
    z = h_ref[...] + a0_ref[...] + a1_ref[...]
    t = jnp.dot(z, w1_ref[...], preferred_element_type=_f32)
    t = jnp.maximum(t + b1_ref[...], 0.0)
    t = jnp.dot(t, w2_ref[...], preferred_element_type=_f32) + b2_ref[...]
    o_ref[...] = jnp.maximum(t, 0.0)


def _pool_heads_body(h_ref, batch_ref, cw1_ref, cb1_ref, cw2_ref, cb2_ref,
                     rw1_ref, rb1_ref, rw2_ref, rb2_ref, s_ref, r_ref):
    b2d = batch_ref[...]
    gids = lax.broadcasted_iota(jnp.int32, (N, G), 1)
    p = (b2d == gids).astype(_f32)
    dims = (((0,), (0,)), ((), ()))
    sums = lax.dot_general(p, h_ref[...], dims,
                           preferred_element_type=_f32)
    counts = lax.dot_general(p, jnp.ones((N, 1), _f32), dims,
                             preferred_element_type=_f32)
    g = sums / jnp.maximum(counts, 1.0)
    cs = jnp.maximum(jnp.dot(g, cw1_ref[...], preferred_element_type=_f32)
                     + cb1_ref[...], 0.0)
    s_ref[...] = (jnp.dot(cs, cw2_ref[...], preferred_element_type=_f32)
                  + cb2_ref[...])
    rs = jnp.maximum(jnp.dot(g, rw1_ref[...], preferred_element_type=_f32)
                     + rb1_ref[...], 0.0)
    r_ref[...] = (jnp.dot(rs, rw2_ref[...], preferred_element_type=_f32)
                  + rb2_ref[...])


def _node_encode(x, w, b):
    return pl.pallas_call(
        _node_encode_body,
        out_shape=jax.ShapeDtypeStruct((N, H), _f32),
    )(x, w, b)


_BE = 2560


def _edge_encode(ea, w1, b1, w2, b2):
    nblk = E // _BE
    return pl.pallas_call(
        _edge_encode_body,
        grid=(nblk,),
        in_specs=[
            pl.BlockSpec((3, _BE), lambda i: (0, i)),
            pl.BlockSpec((3, H), lambda i: (0, 0)),
            pl.BlockSpec((1, H), lambda i: (0, 0)),
            pl.BlockSpec((H, H), lambda i: (0, 0)),
            pl.BlockSpec((1, H), lambda i: (0, 0)),
        ],
        out_specs=pl.BlockSpec((_BE, H), lambda i: (i, 0)),
        out_shape=jax.ShapeDtypeStruct((E, H), _f32),
    )(ea, w1, b1, w2, b2)


_BN = 2000


def _node_mlp(h, a0, a1, w1, b1, w2, b2):
    nblk = N // _BN
    return pl.pallas_call(
        _node_mlp_body,
        grid=(nblk,),
        in_specs=[
            pl.BlockSpec((_BN, H), lambda i: (i, 0)),
            pl.BlockSpec((_BN, H), lambda i: (i, 0)),
            pl.BlockSpec((_BN, H), lambda i: (i, 0)),
            pl.BlockSpec((H, H), lambda i: (0, 0)),
            pl.BlockSpec((1, H), lambda i: (0, 0)),
            pl.BlockSpec((H, H), lambda i: (0, 0)),
            pl.BlockSpec((1, H), lambda i: (0, 0)),
        ],
        out_specs=pl.BlockSpec((_BN, H), lambda i: (i, 0)),
        out_shape=jax.ShapeDtypeStruct((N, H), _f32),
    )(h, a0, a1, w1, b1, w2, b2)


def _pool_heads(h, batch2d, cw1, cb1, cw2, cb2, rw1, rb1, rw2, rb2):
    return pl.pallas_call(
        _pool_heads_body,
        out_shape=(jax.ShapeDtypeStruct((G, 1), _f32),
                   jax.ShapeDtypeStruct((G, 1), _f32)),
    )(h, batch2d, cw1, cb1, cw2, cb2, rw1, rb1, rw2, rb2)


def kernel(x, edge_index, edge_attr, batch,
           ne_w, ne_b, ee_w1, ee_b1, ee_w2, ee_b2,
           conv_w1, conv_b1, conv_w2, conv_b2,
           ch_w1, ch_b1, ch_w2, ch_b2,
           rh_w1, rh_b1, rh_w2, rh_b2):
    src = edge_index[0]
    dst = edge_index[1]
    zeros = jnp.zeros((NPAD, H), _f32)

    h = _node_encode(x, ne_w, ne_b.reshape(1, H))
    ep = _edge_encode(edge_attr.T, ee_w1, ee_b1.reshape(1, H),
                      ee_w2, ee_b2.reshape(1, H))

    for i in range(L):
        parts = _edge_stage(h, ep, src, dst, zeros)
        h = _node_mlp(h, parts[0, :N], parts[1, :N],
                      conv_w1[i], conv_b1[i].reshape(1, H),
                      conv_w2[i], conv_b2[i].reshape(1, H))

    s, r = _pool_heads(h, batch.reshape(N, 1),
                       ch_w1, ch_b1.reshape(1, 64), ch_w2, ch_b2.reshape(1, 1),
                       rh_w1, rh_b1.reshape(1, 64), rh_w2, rh_b2.reshape(1, 1))
    return (s.reshape(G), r.reshape(G))
